# baseline (speedup 1.0000x reference)
import functools

import jax
import jax.numpy as jnp
from jax import lax
from jax.experimental import pallas as pl
from jax.experimental.pallas import tpu as pltpu
from jax.experimental.pallas import tpu_sc as plsc

NC = 2
NS = 16
NW = NC * NS

CHUNK = 128
GROUP = 10
NBUF = 2


def _make_emb(B, V, D):
    assert B % (NW * CHUNK * GROUP) == 0
    bpw = B // NW
    nch = bpw // CHUNK
    ngrp = nch // GROUP
    gsz = GROUP * CHUNK

    mesh = plsc.VectorSubcoreMesh(core_axis_name="c", subcore_axis_name="s")

    @functools.partial(
        pl.kernel,
        mesh=mesh,
        out_type=jax.ShapeDtypeStruct((B, D), jnp.float32),
        scratch_types=[
            pltpu.VMEM((bpw,), jnp.int32),
            pltpu.VMEM((NBUF * gsz, D), jnp.float32),
            pltpu.SemaphoreType.DMA((NBUF,)),
            pltpu.SemaphoreType.DMA((NBUF,)),
        ],
        compiler_params=pltpu.CompilerParams(use_tc_tiling_on_sc=False),
    )
    def emb(idx_hbm, table_hbm, out_hbm, idx_v, rows_v, gsem, wsem):
        wid = lax.axis_index("s") * NC + lax.axis_index("c")
        base = wid * bpw
        pltpu.sync_copy(idx_hbm.at[pl.ds(base, bpw)], idx_v)

        def fire(g, buf):
            for b in range(GROUP):
                pltpu.async_copy(
                    table_hbm.at[idx_v.at[pl.ds((g * GROUP + b) * CHUNK, CHUNK)]],
                    rows_v.at[pl.ds(buf * gsz + b * CHUNK, CHUNK)],
                    gsem.at[buf],
                )

        fire(0, 0)

        def body(g, _):
            cur = g % 2
            nxt = 1 - cur

            @pl.when(g + 1 < ngrp)
            def _():
                @pl.when(g > 0)
                def _():
                    pltpu.make_async_copy(
                        rows_v.at[pl.ds(nxt * gsz, gsz)],
                        out_hbm.at[pl.ds(base, gsz)],
                        wsem.at[nxt],
                    ).wait()
                fire(g + 1, nxt)

            pltpu.make_async_copy(
                out_hbm.at[pl.ds(0, gsz)],
                rows_v.at[pl.ds(cur * gsz, gsz)],
                gsem.at[cur],
            ).wait()
            pltpu.async_copy(
                rows_v.at[pl.ds(cur * gsz, gsz)],
                out_hbm.at[pl.ds(base + g * gsz, gsz)],
                wsem.at[cur],
            )
            return ()

        lax.fori_loop(0, ngrp, body, ())

        for buf in range(NBUF):
            pltpu.make_async_copy(
                rows_v.at[pl.ds(buf * gsz, gsz)],
                out_hbm.at[pl.ds(base, gsz)],
                wsem.at[buf],
            ).wait()

    return emb


def kernel(x, embedding_matrix):
    bat, hist = x.shape
    V, D = embedding_matrix.shape
    B = bat * hist
    idx = x.reshape(-1).astype(jnp.int32)
    out = _make_emb(B, V, D)(idx, embedding_matrix)
    return out.reshape(bat, hist, D)


# speedup vs baseline: 1.2464x; 1.2464x over previous
import functools

import jax
import jax.numpy as jnp
from jax import lax
from jax.experimental import pallas as pl
from jax.experimental.pallas import tpu as pltpu
from jax.experimental.pallas import tpu_sc as plsc

NC = 2
NS = 16
NW = NC * NS

RG = 16
NBUF = 2


def _make_emb(BAT, HIST, V, D):
    assert BAT % NW == 0
    rpw = BAT // NW
    assert rpw % RG == 0
    ngrp = rpw // RG

    mesh = plsc.VectorSubcoreMesh(core_axis_name="c", subcore_axis_name="s")

    @functools.partial(
        pl.kernel,
        mesh=mesh,
        out_type=jax.ShapeDtypeStruct((BAT, HIST, D), jnp.float32),
        scratch_types=[
            pltpu.VMEM((rpw, HIST), jnp.int32),
            pltpu.VMEM((NBUF * RG, HIST, D), jnp.float32),
            pltpu.SemaphoreType.DMA((NBUF,)),
            pltpu.SemaphoreType.DMA((NBUF,)),
        ],
        compiler_params=pltpu.CompilerParams(use_tc_tiling_on_sc=False),
    )
    def emb(x_hbm, table_hbm, out_hbm, idx_v, rows_v, gsem, wsem):
        wid = lax.axis_index("s") * NC + lax.axis_index("c")
        base = wid * rpw
        pltpu.sync_copy(x_hbm.at[pl.ds(base, rpw)], idx_v)

        def fire(g, buf):
            for k in range(RG):
                pltpu.async_copy(
                    table_hbm.at[idx_v.at[g * RG + k]],
                    rows_v.at[buf * RG + k],
                    gsem.at[buf],
                )

        fire(0, 0)

        def body(g, _):
            cur = g % 2
            nxt = 1 - cur

            @pl.when(g + 1 < ngrp)
            def _():
                @pl.when(g > 0)
                def _():
                    pltpu.make_async_copy(
                        rows_v.at[pl.ds(nxt * RG, RG)],
                        out_hbm.at[pl.ds(base, RG)],
                        wsem.at[nxt],
                    ).wait()
                fire(g + 1, nxt)

            pltpu.make_async_copy(
                out_hbm.at[pl.ds(0, RG)],
                rows_v.at[pl.ds(cur * RG, RG)],
                gsem.at[cur],
            ).wait()
            pltpu.async_copy(
                rows_v.at[pl.ds(cur * RG, RG)],
                out_hbm.at[pl.ds(base + g * RG, RG)],
                wsem.at[cur],
            )
            return ()

        lax.fori_loop(0, ngrp, body, ())

        for buf in range(NBUF):
            pltpu.make_async_copy(
                rows_v.at[pl.ds(buf * RG, RG)],
                out_hbm.at[pl.ds(base, RG)],
                wsem.at[buf],
            ).wait()

    return emb


def kernel(x, embedding_matrix):
    bat, hist = x.shape
    V, D = embedding_matrix.shape
    f = 2 if (hist * 2 <= 128 and bat % (2 * NW) == 0) else 1
    xq = x.reshape(bat // f, hist * f).astype(jnp.int32)
    out = _make_emb(bat // f, hist * f, V, D)(xq, embedding_matrix)
    return out.reshape(bat, hist, D)


# speedup vs baseline: 1.6244x; 1.3032x over previous
import functools

import jax
import jax.numpy as jnp
from jax import lax
from jax.experimental import pallas as pl
from jax.experimental.pallas import tpu as pltpu
from jax.experimental.pallas import tpu_sc as plsc

NC = 2
NS = 16
NW = NC * NS

RG = 32
NBUF = 2


def _make_emb(BAT, HIST, V, D):
    assert BAT % NW == 0
    rpw = BAT // NW
    assert rpw % RG == 0
    ngrp = rpw // RG

    mesh = plsc.VectorSubcoreMesh(core_axis_name="c", subcore_axis_name="s")

    @functools.partial(
        pl.kernel,
        mesh=mesh,
        out_type=jax.ShapeDtypeStruct((BAT, HIST, D), jnp.float32),
        scratch_types=[
            pltpu.VMEM((rpw, HIST), jnp.int32),
            pltpu.VMEM((NBUF * RG, HIST, D), jnp.float32),
            pltpu.SemaphoreType.DMA((NBUF,)),
            pltpu.SemaphoreType.DMA((NBUF,)),
        ],
        compiler_params=pltpu.CompilerParams(use_tc_tiling_on_sc=False),
    )
    def emb(x_hbm, table_hbm, out_hbm, idx_v, rows_v, gsem, wsem):
        wid = lax.axis_index("s") * NC + lax.axis_index("c")
        base = wid * rpw
        pltpu.sync_copy(x_hbm.at[pl.ds(base, rpw)], idx_v)

        def fire(g, buf):
            for k in range(RG):
                pltpu.async_copy(
                    table_hbm.at[idx_v.at[g * RG + k]],
                    rows_v.at[buf * RG + k],
                    gsem.at[buf],
                )

        fire(0, 0)

        def body(g, _):
            cur = g % 2
            nxt = 1 - cur

            @pl.when(g + 1 < ngrp)
            def _():
                @pl.when(g > 0)
                def _():
                    pltpu.make_async_copy(
                        rows_v.at[pl.ds(nxt * RG, RG)],
                        out_hbm.at[pl.ds(base, RG)],
                        wsem.at[nxt],
                    ).wait()
                fire(g + 1, nxt)

            pltpu.make_async_copy(
                out_hbm.at[pl.ds(0, RG)],
                rows_v.at[pl.ds(cur * RG, RG)],
                gsem.at[cur],
            ).wait()
            pltpu.async_copy(
                rows_v.at[pl.ds(cur * RG, RG)],
                out_hbm.at[pl.ds(base + g * RG, RG)],
                wsem.at[cur],
            )
            return ()

        lax.fori_loop(0, ngrp, body, ())

        for buf in range(NBUF):
            pltpu.make_async_copy(
                rows_v.at[pl.ds(buf * RG, RG)],
                out_hbm.at[pl.ds(base, RG)],
                wsem.at[buf],
            ).wait()

    return emb


def kernel(x, embedding_matrix):
    bat, hist = x.shape
    V, D = embedding_matrix.shape
    return _make_emb(bat, hist, V, D)(x.astype(jnp.int32), embedding_matrix)
